# Initial kernel scaffold; baseline (speedup 1.0000x reference)
import jax
import jax.numpy as jnp
from jax.experimental import pallas as pl


def kernel(x, token_table, pos_table):
    raise NotImplementedError("write your pallas kernel here")


# speedup vs baseline: 1.0255x; 1.0255x over previous
import functools

import jax
import jax.numpy as jnp
from jax import lax
from jax.experimental import pallas as pl
from jax.experimental.pallas import tpu as pltpu
from jax.experimental.pallas import tpu_sc as plsc

MAXLEN = 200
EMBED_DIM = 32
LANES = 16
NC = 2
NS = 16
NW = NC * NS

SUB = 100
NSUB = 8
CHUNK = SUB * NSUB


@functools.partial(jax.jit, static_argnums=(3,))
def _run(token_table, idx, pos_tiled, B):
    b_per_w = B // NW
    n_chunks = b_per_w // CHUNK
    mesh = plsc.VectorSubcoreMesh(
        core_axis_name="c", subcore_axis_name="s", num_cores=NC, num_subcores=NS
    )

    @functools.partial(
        pl.kernel,
        out_type=jax.ShapeDtypeStruct((B // SUB, SUB, EMBED_DIM), jnp.float32),
        mesh=mesh,
        scratch_types=[
            pltpu.VMEM((NSUB, SUB), jnp.int32),
            pltpu.VMEM((NSUB, SUB, EMBED_DIM), jnp.float32),
            pltpu.VMEM((NSUB, SUB, EMBED_DIM), jnp.float32),
            pltpu.SemaphoreType.DMA,
        ],
        compiler_params=pltpu.CompilerParams(use_tc_tiling_on_sc=False),
    )
    def k(tok_hbm, idx_hbm, pos_hbm, out_hbm, idx_v, rows_v, pos_v, sem):
        wid = lax.axis_index("s") * NC + lax.axis_index("c")
        row0 = wid * (b_per_w // SUB)
        pltpu.sync_copy(pos_hbm, pos_v)

        def chunk_body(g, carry):
            rbase = row0 + g * NSUB
            pltpu.sync_copy(idx_hbm.at[pl.ds(rbase, NSUB)], idx_v)
            handles = [
                pltpu.async_copy(tok_hbm.at[idx_v.at[j]], rows_v.at[j], sem)
                for j in range(NSUB)
            ]
            for h in handles:
                h.wait()

            def add_body(j, c):
                for r in range(SUB):
                    for half in range(2):
                        sl = pl.ds(half * LANES, LANES)
                        rows_v[j, r, sl] = rows_v[j, r, sl] + pos_v[j, r, sl]
                return c

            lax.fori_loop(0, NSUB, add_body, 0)
            pltpu.sync_copy(rows_v, out_hbm.at[pl.ds(rbase, NSUB)])
            return carry

        lax.fori_loop(0, n_chunks, chunk_body, 0)

    return k(token_table, idx, pos_tiled)


def kernel(x, token_table, pos_table):
    B_rows, S = x.shape
    B = B_rows * S
    idx = x.reshape(B // SUB, SUB).astype(jnp.int32)
    pos_tiled = jnp.tile(pos_table, (CHUNK // MAXLEN, 1)).reshape(
        NSUB, SUB, EMBED_DIM
    )
    out = _run(token_table, idx, pos_tiled, B)
    return out.reshape(B_rows, S, EMBED_DIM)


# speedup vs baseline: 1.0638x; 1.0374x over previous
import functools

import jax
import jax.numpy as jnp
from jax import lax
from jax.experimental import pallas as pl
from jax.experimental.pallas import tpu as pltpu
from jax.experimental.pallas import tpu_sc as plsc

MAXLEN = 200
EMBED_DIM = 32
LANES = 16
NC = 2
NS = 16
NW = NC * NS

SUB = 100
NSUB = 8
CHUNK = SUB * NSUB


@functools.partial(jax.jit, static_argnums=(3,))
def _run(token_table, idx, pos_tiled, B):
    b_per_w = B // NW
    n_chunks = b_per_w // CHUNK
    assert n_chunks % 2 == 0
    mesh = plsc.VectorSubcoreMesh(
        core_axis_name="c", subcore_axis_name="s", num_cores=NC, num_subcores=NS
    )

    @functools.partial(
        pl.kernel,
        out_type=jax.ShapeDtypeStruct((B // SUB, SUB, EMBED_DIM), jnp.float32),
        mesh=mesh,
        scratch_types=[
            pltpu.VMEM((2, NSUB, SUB), jnp.int32),
            pltpu.VMEM((2, NSUB, SUB, EMBED_DIM), jnp.float32),
            pltpu.VMEM((NSUB, SUB, EMBED_DIM), jnp.float32),
            pltpu.SemaphoreType.DMA,
            pltpu.SemaphoreType.DMA,
            pltpu.SemaphoreType.DMA,
            pltpu.SemaphoreType.DMA,
        ],
        compiler_params=pltpu.CompilerParams(use_tc_tiling_on_sc=False),
    )
    def k(tok_hbm, idx_hbm, pos_hbm, out_hbm, idx_v, rows_v, pos_v, g0, g1, s0, s1):
        gsem = (g0, g1)
        ssem = (s0, s1)
        wid = lax.axis_index("s") * NC + lax.axis_index("c")
        row0 = wid * (b_per_w // SUB)
        pltpu.sync_copy(pos_hbm, pos_v)

        def stage(g, buf):
            rbase = row0 + g * NSUB
            pltpu.sync_copy(idx_hbm.at[pl.ds(rbase, NSUB)], idx_v.at[buf])
            for j in range(NSUB):
                pltpu.async_copy(
                    tok_hbm.at[idx_v.at[buf, j]], rows_v.at[buf, j], gsem[buf]
                )

        def drain_gathers(g, buf):
            rbase = row0 + g * NSUB
            for j in range(NSUB):
                pltpu.make_async_copy(
                    tok_hbm.at[idx_v.at[buf, j]], rows_v.at[buf, j], gsem[buf]
                ).wait()

        def process(g, buf):
            drain_gathers(g, buf)

            def add_body(j, c):
                for r in range(SUB):
                    for half in range(2):
                        sl = pl.ds(half * LANES, LANES)
                        rows_v[buf, j, r, sl] = rows_v[buf, j, r, sl] + pos_v[j, r, sl]
                return c

            lax.fori_loop(0, NSUB, add_body, 0)
            rbase = row0 + g * NSUB
            pltpu.async_copy(rows_v.at[buf], out_hbm.at[pl.ds(rbase, NSUB)], ssem[buf])

        def wait_scatter(buf):
            pltpu.make_async_copy(
                rows_v.at[buf], out_hbm.at[pl.ds(row0, NSUB)], ssem[buf]
            ).wait()

        stage(0, 0)

        def pair_body(i, carry):
            for b in range(2):
                g = i * 2 + b
                nb = 1 - b

                @pl.when(g + 1 < n_chunks)
                def _prefetch():
                    @pl.when(g >= 1)
                    def _reuse():
                        wait_scatter(nb)

                    stage(g + 1, nb)

                process(g, b)
            return carry

        lax.fori_loop(0, n_chunks // 2, pair_body, 0)
        wait_scatter(0)
        wait_scatter(1)

    return k(token_table, idx, pos_tiled)


def kernel(x, token_table, pos_table):
    B_rows, S = x.shape
    B = B_rows * S
    idx = x.reshape(B // SUB, SUB).astype(jnp.int32)
    pos_tiled = jnp.tile(pos_table, (CHUNK // MAXLEN, 1)).reshape(
        NSUB, SUB, EMBED_DIM
    )
    out = _run(token_table, idx, pos_tiled, B)
    return out.reshape(B_rows, S, EMBED_DIM)


# speedup vs baseline: 1.4049x; 1.3206x over previous
import functools

import jax
import jax.numpy as jnp
from jax import lax
from jax.experimental import pallas as pl
from jax.experimental.pallas import tpu as pltpu
from jax.experimental.pallas import tpu_sc as plsc

MAXLEN = 200
EMBED_DIM = 32
LANES = 16
NC = 2
NS = 16
NW = NC * NS

BT = 128
NBT = 4
CB = BT * NBT
DT = EMBED_DIM // 8


@jax.jit
def _run(token_table, xT, pos_table):
    S, Bb = xT.shape
    n_chunks_total = S * (Bb // CB)
    n_per_w = n_chunks_total // NW
    assert n_per_w % 2 == 0
    bt_per_s = Bb // BT
    out_words = Bb * S * EMBED_DIM
    s_words = Bb * EMBED_DIM
    dt_words = 8 * Bb
    mesh = plsc.VectorSubcoreMesh(
        core_axis_name="c", subcore_axis_name="s", num_cores=NC, num_subcores=NS
    )

    @functools.partial(
        pl.kernel,
        out_type=jax.ShapeDtypeStruct((out_words,), jnp.float32),
        mesh=mesh,
        scratch_types=[
            pltpu.VMEM((2, CB), jnp.int32),
            pltpu.VMEM((2, NBT, BT, EMBED_DIM), jnp.float32),
            pltpu.VMEM((2, NBT * BT * EMBED_DIM), jnp.float32),
            pltpu.VMEM((MAXLEN, EMBED_DIM), jnp.float32),
            pltpu.SemaphoreType.DMA,
            pltpu.SemaphoreType.DMA,
            pltpu.SemaphoreType.DMA,
            pltpu.SemaphoreType.DMA,
        ],
        compiler_params=pltpu.CompilerParams(
            use_tc_tiling_on_sc=False, needs_layout_passes=False
        ),
    )
    def k(tok_hbm, x_hbm, pos_hbm, out_hbm, idx_v, grows_v, obuf_v, pos_v,
          g0, g1, s0, s1):
        gsem = (g0, g1)
        ssem = (s0, s1)
        wid = lax.axis_index("s") * NC + lax.axis_index("c")
        c0 = wid * n_per_w
        pltpu.sync_copy(pos_hbm, pos_v)

        d16 = lax.broadcasted_iota(jnp.int32, (LANES,), 0)
        vb0 = (
            lax.shift_right_logical(d16, 3) * (NBT * 8 * BT)
            + lax.bitwise_and(d16, 7) * BT
        )

        def stage(c, buf):
            s = c // bt_per_s_g
            b0 = (c % bt_per_s_g) * CB
            pltpu.sync_copy(x_hbm.at[s, pl.ds(b0, CB)], idx_v.at[buf])
            for j in range(NBT):
                pltpu.async_copy(
                    tok_hbm.at[idx_v.at[buf, pl.ds(j * BT, BT)]],
                    grows_v.at[buf, j],
                    gsem[buf],
                )

        bt_per_s_g = bt_per_s // NBT

        def drain_gathers(buf):
            for j in range(NBT):
                pltpu.make_async_copy(
                    tok_hbm.at[idx_v.at[buf, pl.ds(j * BT, BT)]],
                    grows_v.at[buf, j],
                    gsem[buf],
                ).wait()

        def process(c, buf):
            drain_gathers(buf)
            s = c // bt_per_s_g
            bt0 = (c % bt_per_s_g) * NBT
            pos0 = pos_v[s, pl.ds(0, LANES)]
            pos1 = pos_v[s, pl.ds(LANES, LANES)]
            ob = obuf_v.at[buf]

            for j in range(NBT):
                base_j = j * BT

                def tr_body(t8, carry):
                    tloc = t8 * 8
                    for u in range(8):
                        t = tloc + u
                        off = base_j * 8 + t
                        idx0 = vb0 + off
                        idx1 = idx0 + 2 * (NBT * 8 * BT)
                        r0 = grows_v[buf, j, t, pl.ds(0, LANES)] + pos0
                        r1 = grows_v[buf, j, t, pl.ds(LANES, LANES)] + pos1
                        plsc.store_scatter(ob, [idx0], r0)
                        plsc.store_scatter(ob, [idx1], r1)
                    return carry

                lax.fori_loop(0, BT // 8, tr_body, 0)

            out_base = s * s_words + bt0 * (8 * BT)
            for dt in range(DT):
                pltpu.async_copy(
                    obuf_v.at[buf, pl.ds(dt * (NBT * 8 * BT), NBT * 8 * BT)],
                    out_hbm.at[pl.ds(out_base + dt * dt_words, NBT * 8 * BT)],
                    ssem[buf],
                )

        def wait_out(buf):
            for dt in range(DT):
                pltpu.make_async_copy(
                    obuf_v.at[buf, pl.ds(dt * (NBT * 8 * BT), NBT * 8 * BT)],
                    out_hbm.at[pl.ds(dt * dt_words, NBT * 8 * BT)],
                    ssem[buf],
                ).wait()

        stage(c0, 0)

        def pair_body(i, carry):
            for b in range(2):
                c = c0 + i * 2 + b
                nb = 1 - b

                @pl.when(i * 2 + b + 1 < n_per_w)
                def _prefetch():
                    stage(c + 1, nb)

                @pl.when(i * 2 + b >= 2)
                def _reuse():
                    wait_out(b)

                process(c, b)
            return carry

        lax.fori_loop(0, n_per_w // 2, pair_body, 0)
        wait_out(0)
        wait_out(1)

    return k(token_table, xT, pos_table)


def kernel(x, token_table, pos_table):
    B_rows, S = x.shape
    xT = x.T.astype(jnp.int32)
    flat = _run(token_table, xT, pos_table)
    r5 = flat.reshape(S, DT, B_rows // BT, 8, BT)
    return r5.transpose(2, 4, 0, 1, 3).reshape(B_rows, S, EMBED_DIM)


# speedup vs baseline: 1.4397x; 1.0248x over previous
import functools

import jax
import jax.numpy as jnp
from jax import lax
from jax.experimental import pallas as pl
from jax.experimental.pallas import tpu as pltpu
from jax.experimental.pallas import tpu_sc as plsc

MAXLEN = 200
EMBED_DIM = 32
LANES = 16
NC = 2
NS = 16
NW = NC * NS

BT = 128
NBT = 4
CB = BT * NBT
DT = EMBED_DIM // 8
OB_WORDS = NBT * 8 * BT


@jax.jit
def _run(token_table, x_flat, pos_table):
    S = MAXLEN
    Bb = x_flat.shape[0] // S
    n_per_w = (S * (Bb // CB)) // NW
    assert n_per_w % 2 == 0
    bt_per_s_g = (Bb // BT) // NBT
    w_words = n_per_w * CB
    out_words = Bb * S * EMBED_DIM
    s_words = Bb * EMBED_DIM
    dt_words = 8 * Bb
    mesh = plsc.VectorSubcoreMesh(
        core_axis_name="c", subcore_axis_name="s", num_cores=NC, num_subcores=NS
    )

    @functools.partial(
        pl.kernel,
        out_type=jax.ShapeDtypeStruct((out_words,), jnp.float32),
        mesh=mesh,
        scratch_types=[
            pltpu.VMEM((w_words,), jnp.int32),
            pltpu.VMEM((2, NBT, BT, EMBED_DIM), jnp.float32),
            pltpu.VMEM((2, DT * OB_WORDS), jnp.float32),
            pltpu.VMEM((MAXLEN, EMBED_DIM), jnp.float32),
            pltpu.SemaphoreType.DMA,
            pltpu.SemaphoreType.DMA,
            pltpu.SemaphoreType.DMA,
            pltpu.SemaphoreType.DMA,
        ],
        compiler_params=pltpu.CompilerParams(
            use_tc_tiling_on_sc=False, needs_layout_passes=False
        ),
    )
    def k(tok_hbm, x_hbm, pos_hbm, out_hbm, idx_v, grows_v, obuf_v, pos_v,
          g0, g1, s0, s1):
        gsem = (g0, g1)
        ssem = (s0, s1)
        wid = lax.axis_index("s") * NC + lax.axis_index("c")
        c0 = wid * n_per_w
        pltpu.sync_copy(pos_hbm, pos_v)
        pltpu.sync_copy(x_hbm.at[pl.ds(c0 * CB, w_words)], idx_v)

        d16 = lax.broadcasted_iota(jnp.int32, (LANES,), 0)
        vb0 = (
            lax.shift_right_logical(d16, 3) * OB_WORDS
            + lax.bitwise_and(d16, 7) * BT
        )
        vbu = [vb0 + u for u in range(LANES)]

        def stage(cloc, buf):
            for j in range(NBT):
                pltpu.async_copy(
                    tok_hbm.at[idx_v.at[pl.ds(cloc * CB + j * BT, BT)]],
                    grows_v.at[buf, j],
                    gsem[buf],
                )

        def drain_gathers(cloc, buf):
            for j in range(NBT):
                pltpu.make_async_copy(
                    tok_hbm.at[idx_v.at[pl.ds(cloc * CB + j * BT, BT)]],
                    grows_v.at[buf, j],
                    gsem[buf],
                ).wait()

        def process(cloc, buf):
            drain_gathers(cloc, buf)
            c = c0 + cloc
            s = c // bt_per_s_g
            bt0 = (c % bt_per_s_g) * NBT
            pos0 = pos_v[s, pl.ds(0, LANES)]
            pos1 = pos_v[s, pl.ds(LANES, LANES)]
            ob = obuf_v.at[buf]

            for j in range(NBT):
                def tr_body(t16, carry, j=j):
                    tloc = t16 * LANES
                    voff = jnp.full((LANES,), j * (8 * BT), jnp.int32) + tloc
                    for u in range(LANES):
                        t = tloc + u
                        i0 = vbu[u] + voff
                        i1 = i0 + 2 * OB_WORDS
                        r0 = grows_v[buf, j, t, pl.ds(0, LANES)] + pos0
                        r1 = grows_v[buf, j, t, pl.ds(LANES, LANES)] + pos1
                        plsc.store_scatter(ob, [i0], r0)
                        plsc.store_scatter(ob, [i1], r1)
                    return carry

                lax.fori_loop(0, BT // LANES, tr_body, 0)

            out_base = s * s_words + bt0 * (8 * BT)
            for dt in range(DT):
                pltpu.async_copy(
                    obuf_v.at[buf, pl.ds(dt * OB_WORDS, OB_WORDS)],
                    out_hbm.at[pl.ds(out_base + dt * dt_words, OB_WORDS)],
                    ssem[buf],
                )

        def wait_out(buf):
            for dt in range(DT):
                pltpu.make_async_copy(
                    obuf_v.at[buf, pl.ds(dt * OB_WORDS, OB_WORDS)],
                    out_hbm.at[pl.ds(dt * dt_words, OB_WORDS)],
                    ssem[buf],
                ).wait()

        stage(0, 0)

        def pair_body(i, carry):
            for b in range(2):
                cloc = i * 2 + b
                nb = 1 - b

                @pl.when(cloc + 1 < n_per_w)
                def _prefetch():
                    stage(cloc + 1, nb)

                @pl.when(cloc >= 2)
                def _reuse():
                    wait_out(b)

                process(cloc, b)
            return carry

        lax.fori_loop(0, n_per_w // 2, pair_body, 0)
        wait_out(0)
        wait_out(1)

    return k(token_table, x_flat, pos_table)


def kernel(x, token_table, pos_table):
    B_rows, S = x.shape
    x_flat = x.T.astype(jnp.int32).reshape(-1)
    flat = _run(token_table, x_flat, pos_table)
    r5 = flat.reshape(S, DT, B_rows // BT, 8, BT)
    return r5.transpose(2, 4, 0, 1, 3).reshape(B_rows, S, EMBED_DIM)


# speedup vs baseline: 2.0368x; 1.4147x over previous
import functools

import jax
import jax.numpy as jnp
from jax import lax
from jax.experimental import pallas as pl
from jax.experimental.pallas import tpu as pltpu
from jax.experimental.pallas import tpu_sc as plsc

MAXLEN = 200
EMBED_DIM = 32
LANES = 16
NC = 2
NS = 16
NW = NC * NS

BT = 128
SR = 4
CB = SR * BT
DT = EMBED_DIM // 8
PITCH = BT + 1
DT_W = 8 * PITCH
SR_W = DT * DT_W


@jax.jit
def _run(token_table, x_flat, pos_table):
    S = MAXLEN
    Bb = x_flat.shape[0] // S
    n_per_w = (S * Bb // CB) // NW
    assert n_per_w % 2 == 0
    t_per_s = Bb // BT
    w_words = n_per_w * CB
    out_words = Bb * S * EMBED_DIM
    mesh = plsc.VectorSubcoreMesh(
        core_axis_name="c", subcore_axis_name="s", num_cores=NC, num_subcores=NS
    )

    @functools.partial(
        pl.kernel,
        out_type=jax.ShapeDtypeStruct(
            (S // 8, 8, DT, Bb // BT, 8, BT), jnp.float32
        ),
        mesh=mesh,
        scratch_types=[
            pltpu.VMEM((w_words,), jnp.int32),
            pltpu.VMEM((2, SR, BT, EMBED_DIM), jnp.float32),
            pltpu.VMEM((2, SR, DT, 8, PITCH), jnp.float32),
            pltpu.VMEM((MAXLEN, EMBED_DIM), jnp.float32),
            pltpu.SemaphoreType.DMA,
            pltpu.SemaphoreType.DMA,
            pltpu.SemaphoreType.DMA,
            pltpu.SemaphoreType.DMA,
        ],
        compiler_params=pltpu.CompilerParams(
            use_tc_tiling_on_sc=False, needs_layout_passes=False
        ),
    )
    def k(tok_hbm, x_hbm, pos_hbm, out_hbm, idx_v, grows_v, obuf_v, pos_v,
          g0, g1, s0, s1):
        gsem = (g0, g1)
        ssem = (s0, s1)
        wid = lax.axis_index("s") * NC + lax.axis_index("c")
        c0 = wid * n_per_w
        pltpu.sync_copy(pos_hbm, pos_v)
        pltpu.sync_copy(x_hbm.at[pl.ds(c0 * CB, w_words)], idx_v)

        d16 = lax.broadcasted_iota(jnp.int32, (LANES,), 0)
        i_dt = lax.shift_right_logical(d16, 3)
        i_dr = lax.bitwise_and(d16, 7)

        def stage(cloc, buf):
            for j in range(SR):
                pltpu.async_copy(
                    tok_hbm.at[idx_v.at[pl.ds(cloc * CB + j * BT, BT)]],
                    grows_v.at[buf, j],
                    gsem[buf],
                )

        def drain_gathers(cloc, buf):
            for j in range(SR):
                pltpu.make_async_copy(
                    tok_hbm.at[idx_v.at[pl.ds(cloc * CB + j * BT, BT)]],
                    grows_v.at[buf, j],
                    gsem[buf],
                ).wait()

        def process(cloc, buf):
            drain_gathers(cloc, buf)
            c = c0 + cloc
            tile = c // 2
            h = c % 2
            sT = tile // t_per_s
            bT = tile % t_per_s
            s_base = sT * 8 + h * SR

            for sr in range(SR):
                pos0 = pos_v[s_base + sr, pl.ds(0, LANES)]
                pos1 = pos_v[s_base + sr, pl.ds(LANES, LANES)]
                ob3 = obuf_v.at[buf, sr]

                def tr_body(b16, carry, sr=sr, ob3=ob3, pos0=pos0, pos1=pos1):
                    tloc = b16 * LANES
                    for u in range(LANES):
                        t = tloc + u
                        i_br = jnp.full((LANES,), 0, jnp.int32) + t
                        r0 = grows_v[buf, sr, t, pl.ds(0, LANES)] + pos0
                        r1 = grows_v[buf, sr, t, pl.ds(LANES, LANES)] + pos1
                        plsc.store_scatter(ob3, [i_dt, i_dr, i_br], r0)
                        plsc.store_scatter(ob3, [i_dt + 2, i_dr, i_br], r1)
                    return carry

                lax.fori_loop(0, BT // LANES, tr_body, 0)

            for sr in range(SR):
                for dt in range(DT):
                    pltpu.async_copy(
                        obuf_v.at[buf, sr, dt, :, pl.ds(0, BT)],
                        out_hbm.at[sT, h * SR + sr, dt, bT],
                        ssem[buf],
                    )

        def wait_out(buf):
            for sr in range(SR):
                for dt in range(DT):
                    pltpu.make_async_copy(
                        obuf_v.at[buf, sr, dt, :, pl.ds(0, BT)],
                        out_hbm.at[0, sr, dt, 0],
                        ssem[buf],
                    ).wait()

        stage(0, 0)

        def pair_body(i, carry):
            for b in range(2):
                cloc = i * 2 + b
                nb = 1 - b

                @pl.when(cloc + 1 < n_per_w)
                def _prefetch():
                    stage(cloc + 1, nb)

                @pl.when(cloc >= 2)
                def _reuse():
                    wait_out(b)

                process(cloc, b)
            return carry

        lax.fori_loop(0, n_per_w // 2, pair_body, 0)
        wait_out(0)
        wait_out(1)

    return k(token_table, x_flat, pos_table)


def kernel(x, token_table, pos_table):
    B_rows, S = x.shape
    x_flat = (
        x.T.astype(jnp.int32)
        .reshape(S // 8, 8, B_rows // BT, BT)
        .transpose(0, 2, 1, 3)
        .reshape(-1)
    )
    out6 = _run(token_table, x_flat, pos_table)
    r5 = out6.reshape(S, DT, B_rows // BT, 8, BT)
    return r5.transpose(2, 4, 0, 1, 3).reshape(B_rows, S, EMBED_DIM)


# speedup vs baseline: 2.0610x; 1.0119x over previous
import functools

import jax
import jax.numpy as jnp
from jax import lax
from jax.experimental import pallas as pl
from jax.experimental.pallas import tpu as pltpu
from jax.experimental.pallas import tpu_sc as plsc

MAXLEN = 200
EMBED_DIM = 32
LANES = 16
NC = 2
NS = 16
NW = NC * NS

BT = 128
NBT = 4
CB = NBT * BT
DT = EMBED_DIM // 8
PITCH = BT + 1


@jax.jit
def _run(token_table, xT, pos_table):
    S, Bb = xT.shape
    n_per_w = (S * Bb // CB) // NW
    assert n_per_w % 2 == 0
    g_per_s = Bb // CB
    mesh = plsc.VectorSubcoreMesh(
        core_axis_name="c", subcore_axis_name="s", num_cores=NC, num_subcores=NS
    )

    @functools.partial(
        pl.kernel,
        out_type=jax.ShapeDtypeStruct((S, DT, Bb // BT, 8, BT), jnp.float32),
        mesh=mesh,
        scratch_types=[
            pltpu.VMEM((2, CB), jnp.int32),
            pltpu.VMEM((2, NBT, BT, EMBED_DIM), jnp.float32),
            pltpu.VMEM((2, NBT, DT, 8, PITCH), jnp.float32),
            pltpu.VMEM((MAXLEN, EMBED_DIM), jnp.float32),
            pltpu.SemaphoreType.DMA,
            pltpu.SemaphoreType.DMA,
            pltpu.SemaphoreType.DMA,
            pltpu.SemaphoreType.DMA,
            pltpu.SemaphoreType.DMA,
            pltpu.SemaphoreType.DMA,
        ],
        compiler_params=pltpu.CompilerParams(
            use_tc_tiling_on_sc=False, needs_layout_passes=False
        ),
    )
    def k(tok_hbm, x_hbm, pos_hbm, out_hbm, idx_v, grows_v, obuf_v, pos_v,
          g0, g1, s0, s1, i0, i1):
        gsem = (g0, g1)
        ssem = (s0, s1)
        isem = (i0, i1)
        wid = lax.axis_index("s") * NC + lax.axis_index("c")
        c0 = wid * n_per_w
        pltpu.sync_copy(pos_hbm, pos_v)

        d16 = lax.broadcasted_iota(jnp.int32, (LANES,), 0)
        i_dt = lax.shift_right_logical(d16, 3)
        i_dr = lax.bitwise_and(d16, 7)

        def idx_start(cloc, buf):
            c = c0 + cloc
            s = c // g_per_s
            b0 = (c % g_per_s) * CB
            pltpu.async_copy(
                x_hbm.at[s, pl.ds(b0, CB)], idx_v.at[buf], isem[buf]
            )

        def idx_wait(buf):
            pltpu.make_async_copy(
                x_hbm.at[0, pl.ds(0, CB)], idx_v.at[buf], isem[buf]
            ).wait()

        def fire_gathers(buf):
            for j in range(NBT):
                pltpu.async_copy(
                    tok_hbm.at[idx_v.at[buf, pl.ds(j * BT, BT)]],
                    grows_v.at[buf, j],
                    gsem[buf],
                )

        def drain_gathers(buf):
            for j in range(NBT):
                pltpu.make_async_copy(
                    tok_hbm.at[idx_v.at[buf, pl.ds(j * BT, BT)]],
                    grows_v.at[buf, j],
                    gsem[buf],
                ).wait()

        def process(cloc, buf):
            c = c0 + cloc
            s = c // g_per_s
            bt0 = (c % g_per_s) * NBT
            pos0 = pos_v[s, pl.ds(0, LANES)]
            pos1 = pos_v[s, pl.ds(LANES, LANES)]

            for btc in range(NBT):
                ob3 = obuf_v.at[buf, btc]

                def tr_body(b16, carry, btc=btc, ob3=ob3):
                    tl = b16 * LANES
                    for u in range(LANES):
                        t = tl + u
                        i_br = jnp.full((LANES,), 0, jnp.int32) + t
                        r0 = grows_v[buf, btc, t, pl.ds(0, LANES)] + pos0
                        r1 = grows_v[buf, btc, t, pl.ds(LANES, LANES)] + pos1
                        plsc.store_scatter(ob3, [i_dt, i_dr, i_br], r0)
                        plsc.store_scatter(ob3, [i_dt + 2, i_dr, i_br], r1)
                    return carry

                lax.fori_loop(0, BT // LANES, tr_body, 0)

            for btc in range(NBT):
                for dt in range(DT):
                    pltpu.async_copy(
                        obuf_v.at[buf, btc, dt, :, pl.ds(0, BT)],
                        out_hbm.at[s, dt, bt0 + btc],
                        ssem[buf],
                    )

        def wait_out(buf):
            for btc in range(NBT):
                for dt in range(DT):
                    pltpu.make_async_copy(
                        obuf_v.at[buf, btc, dt, :, pl.ds(0, BT)],
                        out_hbm.at[0, dt, btc],
                        ssem[buf],
                    ).wait()

        idx_start(0, 0)
        idx_wait(0)
        fire_gathers(0)
        idx_start(1, 1)

        def pair_body(i, carry):
            for b in range(2):
                cloc = i * 2 + b
                nb = 1 - b
                drain_gathers(b)

                @pl.when(cloc + 2 < n_per_w)
                def _pref_idx():
                    idx_start(cloc + 2, b)

                @pl.when(cloc + 1 < n_per_w)
                def _pref_gather():
                    idx_wait(nb)
                    fire_gathers(nb)

                @pl.when(cloc >= 2)
                def _reuse():
                    wait_out(b)

                process(cloc, b)
            return carry

        lax.fori_loop(0, n_per_w // 2, pair_body, 0)
        wait_out(0)
        wait_out(1)

    return k(token_table, xT, pos_table)


def kernel(x, token_table, pos_table):
    B_rows, S = x.shape
    out5 = _run(token_table, x.T.astype(jnp.int32), pos_table)
    return out5.transpose(2, 4, 0, 1, 3).reshape(B_rows, S, EMBED_DIM)
